# Initial kernel scaffold; baseline (speedup 1.0000x reference)
import jax
import jax.numpy as jnp
from jax.experimental import pallas as pl


def kernel(newpc, originpc):
    raise NotImplementedError("write your pallas kernel here")


# speedup vs baseline: 1.1401x; 1.1401x over previous
import functools

import jax
import jax.numpy as jnp
from jax import lax
from jax.experimental import pallas as pl
from jax.experimental.pallas import tpu as pltpu

_STEPLIM = 10
_TOL = 0.0001
_B = 8
_N = 2048
_SUB = 16
_LANE = 128
_UNROLL = 8


def _stats_kernel(o_ref, t_smem, t_ref, out_ref):
    ox = o_ref[0, 0]
    oy = o_ref[0, 1]
    oz = o_ref[0, 2]
    tx = t_ref[0, 0]
    ty = t_ref[0, 1]
    tz = t_ref[0, 2]

    def jbody(jc, acc):
        for u in range(_UNROLL):
            j = jc * _UNROLL + u
            sx = t_smem[0, 0, j]
            sy = t_smem[0, 1, j]
            sz = t_smem[0, 2, j]
            dx = ox - sx
            dy = oy - sy
            dz = oz - sz
            acc = jnp.minimum(acc, dx * dx + dy * dy + dz * dz)
        return acc

    acc0 = jnp.full((_SUB, _LANE), jnp.inf, dtype=jnp.float32)
    acc = lax.fori_loop(0, _N // _UNROLL, jbody, acc0)
    out_ref[0, 0, 0] = jnp.sum(jnp.sqrt(acc))

    inv_n = jnp.float32(1.0 / _N)
    c1 = (jnp.sum(tx) * inv_n, jnp.sum(ty) * inv_n, jnp.sum(tz) * inv_n)
    c2 = (jnp.sum(ox) * inv_n, jnp.sum(oy) * inv_n, jnp.sum(oz) * inv_n)
    for k in range(3):
        out_ref[0, 0, 1 + k] = c1[k]
        out_ref[0, 0, 4 + k] = c2[k]
    po = (ox - c2[0], oy - c2[1], oz - c2[2])
    pt = (tx - c1[0], ty - c1[1], tz - c1[2])
    for a in range(3):
        for b in range(3):
            out_ref[0, 0, 7 + 3 * a + b] = jnp.sum(po[a] * pt[b])


def _cov_kernel(o_ref, t_ref, out_ref):
    ox = o_ref[0, 0]
    oy = o_ref[0, 1]
    oz = o_ref[0, 2]
    tx = t_ref[0, 0]
    ty = t_ref[0, 1]
    tz = t_ref[0, 2]
    out_ref[0, 0, 0] = jnp.float32(0.0)
    inv_n = jnp.float32(1.0 / _N)
    c1 = (jnp.sum(tx) * inv_n, jnp.sum(ty) * inv_n, jnp.sum(tz) * inv_n)
    c2 = (jnp.sum(ox) * inv_n, jnp.sum(oy) * inv_n, jnp.sum(oz) * inv_n)
    for k in range(3):
        out_ref[0, 0, 1 + k] = c1[k]
        out_ref[0, 0, 4 + k] = c2[k]
    po = (ox - c2[0], oy - c2[1], oz - c2[2])
    pt = (tx - c1[0], ty - c1[1], tz - c1[2])
    for a in range(3):
        for b in range(3):
            out_ref[0, 0, 7 + 3 * a + b] = jnp.sum(po[a] * pt[b])


def _loop_stats(o4, t4, t_flat):
    return pl.pallas_call(
        _stats_kernel,
        grid=(_B,),
        in_specs=[
            pl.BlockSpec((1, 3, _SUB, _LANE), lambda b: (b, 0, 0, 0)),
            pl.BlockSpec((1, 3, _N), lambda b: (b, 0, 0),
                         memory_space=pltpu.SMEM),
            pl.BlockSpec((1, 3, _SUB, _LANE), lambda b: (b, 0, 0, 0)),
        ],
        out_specs=pl.BlockSpec((1, 1, 16), lambda b: (b, 0, 0),
                               memory_space=pltpu.SMEM),
        out_shape=jax.ShapeDtypeStruct((_B, 1, 16), jnp.float32),
    )(o4, t_flat, t4)


def _final_stats(o4, t4):
    return pl.pallas_call(
        _cov_kernel,
        grid=(_B,),
        in_specs=[
            pl.BlockSpec((1, 3, _SUB, _LANE), lambda b: (b, 0, 0, 0)),
            pl.BlockSpec((1, 3, _SUB, _LANE), lambda b: (b, 0, 0, 0)),
        ],
        out_specs=pl.BlockSpec((1, 1, 16), lambda b: (b, 0, 0),
                               memory_space=pltpu.SMEM),
        out_shape=jax.ShapeDtypeStruct((_B, 1, 16), jnp.float32),
    )(o4, t4)


def _kabsch(stats):
    c1 = stats[:, 1:4]
    c2 = stats[:, 4:7]
    M = stats[:, 7:16].reshape(_B, 3, 3)
    U, S, Vh = jnp.linalg.svd(M, full_matrices=False)
    s = jnp.linalg.det(jnp.matmul(U, Vh))
    ones = jnp.ones_like(s)
    D = jnp.stack([ones, ones, s], axis=-1)
    R = jnp.matmul(U * D[..., None, :], Vh)
    t = c2 - jnp.einsum('bij,bj->bi', R, c1)
    return R, t


def kernel(newpc, originpc):
    oc = jnp.swapaxes(originpc, -1, -2)
    o4 = oc.reshape(_B, 3, _SUB, _LANE)
    nc = jnp.swapaxes(newpc, -1, -2)

    def cond_fun(state):
        it, err, tc, done = state
        return jnp.logical_and(it <= _STEPLIM, jnp.logical_not(done))

    def body_fun(state):
        it, err, tc, done = state
        it = it + 1
        stats = _loop_stats(o4, tc.reshape(_B, 3, _SUB, _LANE), tc)[:, 0]
        errnew = jnp.sum(stats[:, 0]) / jnp.float32(_B * _N)
        R, t = _kabsch(stats)
        tc = jnp.einsum('bij,bjn->bin', R, tc) + t[:, :, None]
        done = jnp.abs(err - errnew) < _TOL
        return it, errnew, tc, done

    it0 = jnp.array(0, dtype=jnp.int32)
    err0 = jnp.array(0.0, dtype=newpc.dtype)
    done0 = jnp.array(False)
    _, _, tc, _ = lax.while_loop(
        cond_fun, body_fun, (it0, err0, nc, done0))

    stats2 = _final_stats(tc.reshape(_B, 3, _SUB, _LANE),
                          nc.reshape(_B, 3, _SUB, _LANE))[:, 0]
    R, t = _kabsch(stats2)
    T = jnp.zeros((_B, 4, 4), dtype=newpc.dtype)
    T = T.at[:, :3, :3].set(R)
    T = T.at[:, :3, 3].set(t)
    T = T.at[:, 3, 3].set(1.0)
    return T


# speedup vs baseline: 2.2846x; 2.0039x over previous
import functools

import jax
import jax.numpy as jnp
from jax import lax
from jax.experimental import pallas as pl
from jax.experimental.pallas import tpu as pltpu

_STEPLIM = 10
_TOL = 0.0001
_B = 8
_N = 2048
_SUB = 16
_LANE = 128
_UNROLL = 8


def _stats_kernel(o_ref, t_smem, t_ref, out_ref):
    ox = o_ref[0, 0]
    oy = o_ref[0, 1]
    oz = o_ref[0, 2]
    tx = t_ref[0, 0]
    ty = t_ref[0, 1]
    tz = t_ref[0, 2]

    def jbody(jc, acc):
        for u in range(_UNROLL):
            j = jc * _UNROLL + u
            sx = t_smem[0, 0, j]
            sy = t_smem[0, 1, j]
            sz = t_smem[0, 2, j]
            dx = ox - sx
            dy = oy - sy
            dz = oz - sz
            acc = jnp.minimum(acc, dx * dx + dy * dy + dz * dz)
        return acc

    acc0 = jnp.full((_SUB, _LANE), jnp.inf, dtype=jnp.float32)
    acc = lax.fori_loop(0, _N // _UNROLL, jbody, acc0)
    out_ref[0, 0, 0] = jnp.sum(jnp.sqrt(acc))

    inv_n = jnp.float32(1.0 / _N)
    c1 = (jnp.sum(tx) * inv_n, jnp.sum(ty) * inv_n, jnp.sum(tz) * inv_n)
    c2 = (jnp.sum(ox) * inv_n, jnp.sum(oy) * inv_n, jnp.sum(oz) * inv_n)
    for k in range(3):
        out_ref[0, 0, 1 + k] = c1[k]
        out_ref[0, 0, 4 + k] = c2[k]
    po = (ox - c2[0], oy - c2[1], oz - c2[2])
    pt = (tx - c1[0], ty - c1[1], tz - c1[2])
    for a in range(3):
        for b in range(3):
            out_ref[0, 0, 7 + 3 * a + b] = jnp.sum(po[a] * pt[b])


def _cov_kernel(o_ref, t_ref, out_ref):
    ox = o_ref[0, 0]
    oy = o_ref[0, 1]
    oz = o_ref[0, 2]
    tx = t_ref[0, 0]
    ty = t_ref[0, 1]
    tz = t_ref[0, 2]
    out_ref[0, 0, 0] = jnp.float32(0.0)
    inv_n = jnp.float32(1.0 / _N)
    c1 = (jnp.sum(tx) * inv_n, jnp.sum(ty) * inv_n, jnp.sum(tz) * inv_n)
    c2 = (jnp.sum(ox) * inv_n, jnp.sum(oy) * inv_n, jnp.sum(oz) * inv_n)
    for k in range(3):
        out_ref[0, 0, 1 + k] = c1[k]
        out_ref[0, 0, 4 + k] = c2[k]
    po = (ox - c2[0], oy - c2[1], oz - c2[2])
    pt = (tx - c1[0], ty - c1[1], tz - c1[2])
    for a in range(3):
        for b in range(3):
            out_ref[0, 0, 7 + 3 * a + b] = jnp.sum(po[a] * pt[b])


def _loop_stats(o4, t4, t_flat):
    return pl.pallas_call(
        _stats_kernel,
        grid=(_B,),
        in_specs=[
            pl.BlockSpec((1, 3, _SUB, _LANE), lambda b: (b, 0, 0, 0)),
            pl.BlockSpec((1, 3, _N), lambda b: (b, 0, 0),
                         memory_space=pltpu.SMEM),
            pl.BlockSpec((1, 3, _SUB, _LANE), lambda b: (b, 0, 0, 0)),
        ],
        out_specs=pl.BlockSpec((1, 1, 16), lambda b: (b, 0, 0),
                               memory_space=pltpu.SMEM),
        out_shape=jax.ShapeDtypeStruct((_B, 1, 16), jnp.float32),
    )(o4, t_flat, t4)


def _final_stats(o4, t4):
    return pl.pallas_call(
        _cov_kernel,
        grid=(_B,),
        in_specs=[
            pl.BlockSpec((1, 3, _SUB, _LANE), lambda b: (b, 0, 0, 0)),
            pl.BlockSpec((1, 3, _SUB, _LANE), lambda b: (b, 0, 0, 0)),
        ],
        out_specs=pl.BlockSpec((1, 1, 16), lambda b: (b, 0, 0),
                               memory_space=pltpu.SMEM),
        out_shape=jax.ShapeDtypeStruct((_B, 1, 16), jnp.float32),
    )(o4, t4)


def _jacobi_kabsch(M, c1, c2):
    m = [[M[:, i, j] for j in range(3)] for i in range(3)]
    a = {}
    for i in range(3):
        for j in range(i, 3):
            a[(i, j)] = sum(m[k][i] * m[k][j] for k in range(3))
    v = {(i, j): jnp.full_like(M[:, 0, 0], 1.0 if i == j else 0.0)
         for i in range(3) for j in range(3)}

    def ga(i, j):
        return a[(i, j)] if i <= j else a[(j, i)]

    for _ in range(8):
        for (p, q) in ((0, 1), (0, 2), (1, 2)):
            apq = ga(p, q)
            app = ga(p, p)
            aqq = ga(q, q)
            denom = 2.0 * apq
            safe = jnp.where(denom == 0.0, 1.0, denom)
            tau = (aqq - app) / safe
            tt = jnp.sign(tau) / (jnp.abs(tau) + jnp.sqrt(1.0 + tau * tau))
            tt = jnp.where(denom == 0.0, 0.0, tt)
            c = 1.0 / jnp.sqrt(1.0 + tt * tt)
            s = tt * c
            r = 3 - p - q
            arp = ga(r, p)
            arq = ga(r, q)
            a[(p, p)] = app - tt * apq
            a[(q, q)] = aqq + tt * apq
            a[(p, q)] = jnp.zeros_like(apq)
            a[(min(r, p), max(r, p))] = c * arp - s * arq
            a[(min(r, q), max(r, q))] = s * arp + c * arq
            for i in range(3):
                vip = v[(i, p)]
                viq = v[(i, q)]
                v[(i, p)] = c * vip - s * viq
                v[(i, q)] = s * vip + c * viq

    w = [a[(0, 0)], a[(1, 1)], a[(2, 2)]]
    cols = [[v[(i, k)] for i in range(3)] for k in range(3)]
    for (x, y) in ((0, 1), (0, 2), (1, 2)):
        sw = w[x] < w[y]
        w[x], w[y] = jnp.where(sw, w[y], w[x]), jnp.where(sw, w[x], w[y])
        for i in range(3):
            cx, cy = cols[x][i], cols[y][i]
            cols[x][i] = jnp.where(sw, cy, cx)
            cols[y][i] = jnp.where(sw, cx, cy)

    def matvec(col):
        return [sum(m[i][j] * col[j] for j in range(3)) for i in range(3)]

    u0 = matvec(cols[0])
    n0 = jnp.sqrt(sum(x * x for x in u0))
    u0 = [x / n0 for x in u0]
    u1 = matvec(cols[1])
    d01 = sum(u0[i] * u1[i] for i in range(3))
    u1 = [u1[i] - d01 * u0[i] for i in range(3)]
    n1 = jnp.sqrt(sum(x * x for x in u1))
    u1 = [x / n1 for x in u1]
    u2 = [u0[1] * u1[2] - u0[2] * u1[1],
          u0[2] * u1[0] - u0[0] * u1[2],
          u0[0] * u1[1] - u0[1] * u1[0]]
    detv = (cols[0][0] * (cols[1][1] * cols[2][2] - cols[2][1] * cols[1][2])
            - cols[1][0] * (cols[0][1] * cols[2][2] - cols[2][1] * cols[0][2])
            + cols[2][0] * (cols[0][1] * cols[1][2] - cols[1][1] * cols[0][2]))
    sgn = jnp.sign(detv)
    u2 = [x * sgn for x in u2]
    U = [u0, u1, u2]
    R = jnp.stack([
        jnp.stack([sum(U[k][i] * cols[k][j] for k in range(3))
                   for j in range(3)], axis=-1)
        for i in range(3)], axis=-2)
    t = c2 - jnp.einsum('bij,bj->bi', R, c1)
    return R, t


def _kabsch(stats):
    c1 = stats[:, 1:4]
    c2 = stats[:, 4:7]
    M = stats[:, 7:16].reshape(_B, 3, 3)
    return _jacobi_kabsch(M, c1, c2)


def kernel(newpc, originpc):
    oc = jnp.swapaxes(originpc, -1, -2)
    o4 = oc.reshape(_B, 3, _SUB, _LANE)
    nc = jnp.swapaxes(newpc, -1, -2)

    def cond_fun(state):
        it, err, tc, done = state
        return jnp.logical_and(it <= _STEPLIM, jnp.logical_not(done))

    def body_fun(state):
        it, err, tc, done = state
        it = it + 1
        stats = _loop_stats(o4, tc.reshape(_B, 3, _SUB, _LANE), tc)[:, 0]
        errnew = jnp.sum(stats[:, 0]) / jnp.float32(_B * _N)
        R, t = _kabsch(stats)
        tc = jnp.einsum('bij,bjn->bin', R, tc) + t[:, :, None]
        done = jnp.abs(err - errnew) < _TOL
        return it, errnew, tc, done

    it0 = jnp.array(0, dtype=jnp.int32)
    err0 = jnp.array(0.0, dtype=newpc.dtype)
    done0 = jnp.array(False)
    _, _, tc, _ = lax.while_loop(
        cond_fun, body_fun, (it0, err0, nc, done0))

    stats2 = _final_stats(tc.reshape(_B, 3, _SUB, _LANE),
                          nc.reshape(_B, 3, _SUB, _LANE))[:, 0]
    R, t = _kabsch(stats2)
    T = jnp.zeros((_B, 4, 4), dtype=newpc.dtype)
    T = T.at[:, :3, :3].set(R)
    T = T.at[:, :3, 3].set(t)
    T = T.at[:, 3, 3].set(1.0)
    return T


# speedup vs baseline: 2.5694x; 1.1247x over previous
import functools

import jax
import jax.numpy as jnp
from jax import lax
from jax.experimental import pallas as pl
from jax.experimental.pallas import tpu as pltpu

_STEPLIM = 10
_TOL = 0.0001
_B = 8
_N = 2048
_SUB = 16
_LANE = 128
_UNROLL = 16
_NACC = 4


def _stats_kernel(o_ref, t_smem, t_ref, out_ref):
    ox = o_ref[0, 0]
    oy = o_ref[0, 1]
    oz = o_ref[0, 2]
    tx = t_ref[0, 0]
    ty = t_ref[0, 1]
    tz = t_ref[0, 2]

    def jbody(jc, accs):
        accs = list(accs)
        for u in range(_UNROLL):
            j = jc * _UNROLL + u
            sx = t_smem[0, 0, j]
            sy = t_smem[0, 1, j]
            sz = t_smem[0, 2, j]
            dx = ox - sx
            dy = oy - sy
            dz = oz - sz
            k = u % _NACC
            accs[k] = jnp.minimum(accs[k], dx * dx + dy * dy + dz * dz)
        return tuple(accs)

    inf0 = jnp.full((_SUB, _LANE), jnp.inf, dtype=jnp.float32)
    accs = lax.fori_loop(0, _N // _UNROLL, jbody, (inf0,) * _NACC)
    acc = jnp.minimum(jnp.minimum(accs[0], accs[1]),
                      jnp.minimum(accs[2], accs[3]))
    out_ref[0, 0, 0] = jnp.sum(jnp.sqrt(acc))

    inv_n = jnp.float32(1.0 / _N)
    c1 = (jnp.sum(tx) * inv_n, jnp.sum(ty) * inv_n, jnp.sum(tz) * inv_n)
    c2 = (jnp.sum(ox) * inv_n, jnp.sum(oy) * inv_n, jnp.sum(oz) * inv_n)
    for k in range(3):
        out_ref[0, 0, 1 + k] = c1[k]
        out_ref[0, 0, 4 + k] = c2[k]
    po = (ox - c2[0], oy - c2[1], oz - c2[2])
    pt = (tx - c1[0], ty - c1[1], tz - c1[2])
    for a in range(3):
        for b in range(3):
            out_ref[0, 0, 7 + 3 * a + b] = jnp.sum(po[a] * pt[b])


def _cov_kernel(o_ref, t_ref, out_ref):
    ox = o_ref[0, 0]
    oy = o_ref[0, 1]
    oz = o_ref[0, 2]
    tx = t_ref[0, 0]
    ty = t_ref[0, 1]
    tz = t_ref[0, 2]
    out_ref[0, 0, 0] = jnp.float32(0.0)
    inv_n = jnp.float32(1.0 / _N)
    c1 = (jnp.sum(tx) * inv_n, jnp.sum(ty) * inv_n, jnp.sum(tz) * inv_n)
    c2 = (jnp.sum(ox) * inv_n, jnp.sum(oy) * inv_n, jnp.sum(oz) * inv_n)
    for k in range(3):
        out_ref[0, 0, 1 + k] = c1[k]
        out_ref[0, 0, 4 + k] = c2[k]
    po = (ox - c2[0], oy - c2[1], oz - c2[2])
    pt = (tx - c1[0], ty - c1[1], tz - c1[2])
    for a in range(3):
        for b in range(3):
            out_ref[0, 0, 7 + 3 * a + b] = jnp.sum(po[a] * pt[b])


def _loop_stats(o4, t4, t_flat):
    return pl.pallas_call(
        _stats_kernel,
        grid=(_B,),
        in_specs=[
            pl.BlockSpec((1, 3, _SUB, _LANE), lambda b: (b, 0, 0, 0)),
            pl.BlockSpec((1, 3, _N), lambda b: (b, 0, 0),
                         memory_space=pltpu.SMEM),
            pl.BlockSpec((1, 3, _SUB, _LANE), lambda b: (b, 0, 0, 0)),
        ],
        out_specs=pl.BlockSpec((1, 1, 16), lambda b: (b, 0, 0),
                               memory_space=pltpu.SMEM),
        out_shape=jax.ShapeDtypeStruct((_B, 1, 16), jnp.float32),
    )(o4, t_flat, t4)


def _final_stats(o4, t4):
    return pl.pallas_call(
        _cov_kernel,
        grid=(_B,),
        in_specs=[
            pl.BlockSpec((1, 3, _SUB, _LANE), lambda b: (b, 0, 0, 0)),
            pl.BlockSpec((1, 3, _SUB, _LANE), lambda b: (b, 0, 0, 0)),
        ],
        out_specs=pl.BlockSpec((1, 1, 16), lambda b: (b, 0, 0),
                               memory_space=pltpu.SMEM),
        out_shape=jax.ShapeDtypeStruct((_B, 1, 16), jnp.float32),
    )(o4, t4)


def _jacobi_kabsch(M, c1, c2):
    m = [[M[:, i, j] for j in range(3)] for i in range(3)]
    a = {}
    for i in range(3):
        for j in range(i, 3):
            a[(i, j)] = sum(m[k][i] * m[k][j] for k in range(3))
    v = {(i, j): jnp.full_like(M[:, 0, 0], 1.0 if i == j else 0.0)
         for i in range(3) for j in range(3)}

    def ga(i, j):
        return a[(i, j)] if i <= j else a[(j, i)]

    for _ in range(8):
        for (p, q) in ((0, 1), (0, 2), (1, 2)):
            apq = ga(p, q)
            app = ga(p, p)
            aqq = ga(q, q)
            denom = 2.0 * apq
            safe = jnp.where(denom == 0.0, 1.0, denom)
            tau = (aqq - app) / safe
            tt = jnp.sign(tau) / (jnp.abs(tau) + jnp.sqrt(1.0 + tau * tau))
            tt = jnp.where(denom == 0.0, 0.0, tt)
            c = 1.0 / jnp.sqrt(1.0 + tt * tt)
            s = tt * c
            r = 3 - p - q
            arp = ga(r, p)
            arq = ga(r, q)
            a[(p, p)] = app - tt * apq
            a[(q, q)] = aqq + tt * apq
            a[(p, q)] = jnp.zeros_like(apq)
            a[(min(r, p), max(r, p))] = c * arp - s * arq
            a[(min(r, q), max(r, q))] = s * arp + c * arq
            for i in range(3):
                vip = v[(i, p)]
                viq = v[(i, q)]
                v[(i, p)] = c * vip - s * viq
                v[(i, q)] = s * vip + c * viq

    w = [a[(0, 0)], a[(1, 1)], a[(2, 2)]]
    cols = [[v[(i, k)] for i in range(3)] for k in range(3)]
    for (x, y) in ((0, 1), (0, 2), (1, 2)):
        sw = w[x] < w[y]
        w[x], w[y] = jnp.where(sw, w[y], w[x]), jnp.where(sw, w[x], w[y])
        for i in range(3):
            cx, cy = cols[x][i], cols[y][i]
            cols[x][i] = jnp.where(sw, cy, cx)
            cols[y][i] = jnp.where(sw, cx, cy)

    def matvec(col):
        return [sum(m[i][j] * col[j] for j in range(3)) for i in range(3)]

    u0 = matvec(cols[0])
    n0 = jnp.sqrt(sum(x * x for x in u0))
    u0 = [x / n0 for x in u0]
    u1 = matvec(cols[1])
    d01 = sum(u0[i] * u1[i] for i in range(3))
    u1 = [u1[i] - d01 * u0[i] for i in range(3)]
    n1 = jnp.sqrt(sum(x * x for x in u1))
    u1 = [x / n1 for x in u1]
    u2 = [u0[1] * u1[2] - u0[2] * u1[1],
          u0[2] * u1[0] - u0[0] * u1[2],
          u0[0] * u1[1] - u0[1] * u1[0]]
    detv = (cols[0][0] * (cols[1][1] * cols[2][2] - cols[2][1] * cols[1][2])
            - cols[1][0] * (cols[0][1] * cols[2][2] - cols[2][1] * cols[0][2])
            + cols[2][0] * (cols[0][1] * cols[1][2] - cols[1][1] * cols[0][2]))
    sgn = jnp.sign(detv)
    u2 = [x * sgn for x in u2]
    U = [u0, u1, u2]
    R = jnp.stack([
        jnp.stack([sum(U[k][i] * cols[k][j] for k in range(3))
                   for j in range(3)], axis=-1)
        for i in range(3)], axis=-2)
    t = c2 - jnp.einsum('bij,bj->bi', R, c1)
    return R, t


def _kabsch(stats):
    c1 = stats[:, 1:4]
    c2 = stats[:, 4:7]
    M = stats[:, 7:16].reshape(_B, 3, 3)
    return _jacobi_kabsch(M, c1, c2)


def kernel(newpc, originpc):
    oc = jnp.swapaxes(originpc, -1, -2)
    o4 = oc.reshape(_B, 3, _SUB, _LANE)
    nc = jnp.swapaxes(newpc, -1, -2)

    def cond_fun(state):
        it, err, tc, done = state
        return jnp.logical_and(it <= _STEPLIM, jnp.logical_not(done))

    def body_fun(state):
        it, err, tc, done = state
        it = it + 1
        stats = _loop_stats(o4, tc.reshape(_B, 3, _SUB, _LANE), tc)[:, 0]
        errnew = jnp.sum(stats[:, 0]) / jnp.float32(_B * _N)
        R, t = _kabsch(stats)
        tc = jnp.einsum('bij,bjn->bin', R, tc) + t[:, :, None]
        done = jnp.abs(err - errnew) < _TOL
        return it, errnew, tc, done

    it0 = jnp.array(0, dtype=jnp.int32)
    err0 = jnp.array(0.0, dtype=newpc.dtype)
    done0 = jnp.array(False)
    _, _, tc, _ = lax.while_loop(
        cond_fun, body_fun, (it0, err0, nc, done0))

    stats2 = _final_stats(tc.reshape(_B, 3, _SUB, _LANE),
                          nc.reshape(_B, 3, _SUB, _LANE))[:, 0]
    R, t = _kabsch(stats2)
    T = jnp.zeros((_B, 4, 4), dtype=newpc.dtype)
    T = T.at[:, :3, :3].set(R)
    T = T.at[:, :3, 3].set(t)
    T = T.at[:, 3, 3].set(1.0)
    return T
